# Initial kernel scaffold; baseline (speedup 1.0000x reference)
import jax
import jax.numpy as jnp
from jax.experimental import pallas as pl


def kernel(word_table, domain_table, x):
    raise NotImplementedError("write your pallas kernel here")


# speedup vs baseline: 1.0358x; 1.0358x over previous
import functools

import jax
import jax.numpy as jnp
from jax import lax
from jax.experimental import pallas as pl
from jax.experimental.pallas import tpu as pltpu
from jax.experimental.pallas import tpu_sc as plsc

WORD_DIM = 300
DOMAIN_DIM = 100
OUT_DIM = WORD_DIM + DOMAIN_DIM
N = 1024 * 200
NUM_WORKERS = 32
N_PER_W = N // NUM_WORKERS
CHUNK = 128
NCHUNKS = N_PER_W // CHUNK

_mesh = plsc.VectorSubcoreMesh(core_axis_name="c", subcore_axis_name="s")


@functools.partial(
    pl.kernel,
    mesh=_mesh,
    out_type=jax.ShapeDtypeStruct((N, OUT_DIM), jnp.float32),
    compiler_params=pltpu.CompilerParams(use_tc_tiling_on_sc=False),
    scratch_types=[
        [pltpu.VMEM((CHUNK,), jnp.int32)] * 2,
        [pltpu.VMEM((CHUNK, OUT_DIM), jnp.float32)] * 2,
        [pltpu.SemaphoreType.DMA] * 2,
        [pltpu.SemaphoreType.DMA] * 2,
    ],
)
def _fused_gather(tab_hbm, idx_hbm, out_hbm, idx_v, rows_v, gsem, ssem):
    wid = lax.axis_index("s") * 2 + lax.axis_index("c")
    base0 = wid * N_PER_W

    def gather(i, slot):
        base = base0 + i * CHUNK
        pltpu.sync_copy(idx_hbm.at[pl.ds(base, CHUNK)], idx_v[slot])
        return pltpu.async_copy(tab_hbm.at[idx_v[slot]], rows_v[slot],
                                gsem[slot])

    def put(i, slot):
        base = base0 + i * CHUNK
        return pltpu.async_copy(rows_v[slot], out_hbm.at[pl.ds(base, CHUNK)],
                                ssem[slot])

    gather(0, 0).wait()
    put(0, 0)
    gather(1, 1).wait()

    def step(i, slot):
        put(i + 1, 1 - slot)
        pltpu.make_async_copy(rows_v[slot], out_hbm.at[pl.ds(0, CHUNK)],
                              ssem[slot]).wait()
        gather(i + 2, slot).wait()

    def body(k, carry):
        step(2 * k, 0)
        step(2 * k + 1, 1)
        return carry

    lax.fori_loop(0, (NCHUNKS - 2) // 2, body, 0)

    last = (NCHUNKS - 1) % 2
    put(NCHUNKS - 1, last)
    pltpu.make_async_copy(rows_v[1 - last], out_hbm.at[pl.ds(0, CHUNK)],
                          ssem[1 - last]).wait()
    pltpu.make_async_copy(rows_v[last], out_hbm.at[pl.ds(0, CHUNK)],
                          ssem[last]).wait()


def kernel(word_table, domain_table, x):
    fused = jnp.concatenate([word_table, domain_table], axis=1)
    idx = x.reshape(-1).astype(jnp.int32)
    out = _fused_gather(fused, idx)
    return out.reshape(x.shape[0], x.shape[1], OUT_DIM)


# speedup vs baseline: 1.3472x; 1.3007x over previous
import functools

import jax
import jax.numpy as jnp
from jax import lax
from jax.experimental import pallas as pl
from jax.experimental.pallas import tpu as pltpu
from jax.experimental.pallas import tpu_sc as plsc

WORD_DIM = 300
DOMAIN_DIM = 100
OUT_DIM = WORD_DIM + DOMAIN_DIM
N = 1024 * 200
NUM_WORKERS = 32
N_PER_W = N // NUM_WORKERS
CHUNK = 128
NCHUNKS = N_PER_W // CHUNK

_mesh = plsc.VectorSubcoreMesh(core_axis_name="c", subcore_axis_name="s")


@functools.partial(
    pl.kernel,
    mesh=_mesh,
    out_type=jax.ShapeDtypeStruct((N, OUT_DIM), jnp.float32),
    compiler_params=pltpu.CompilerParams(use_tc_tiling_on_sc=False),
    scratch_types=[
        [pltpu.VMEM((CHUNK,), jnp.int32)] * 2,
        [pltpu.VMEM((CHUNK, OUT_DIM), jnp.float32)] * 2,
        [pltpu.SemaphoreType.DMA] * 2,
        [pltpu.SemaphoreType.DMA] * 2,
    ],
)
def _fused_gather(tab_hbm, idx_hbm, out_hbm, idx_v, rows_v, gsem, ssem):
    wid = lax.axis_index("s") * 2 + lax.axis_index("c")
    base0 = wid * N_PER_W

    def gather(i, slot):
        base = base0 + i * CHUNK
        pltpu.sync_copy(idx_hbm.at[pl.ds(base, CHUNK)], idx_v[slot])
        return pltpu.async_copy(tab_hbm.at[idx_v[slot]], rows_v[slot],
                                gsem[slot])

    def put(i, slot):
        base = base0 + i * CHUNK
        return pltpu.async_copy(rows_v[slot], out_hbm.at[pl.ds(base, CHUNK)],
                                ssem[slot])

    gather(0, 0).wait()
    put(0, 0)
    gather(1, 1).wait()

    def step(i, slot):
        put(i + 1, 1 - slot)
        pltpu.make_async_copy(rows_v[slot], out_hbm.at[pl.ds(0, CHUNK)],
                              ssem[slot]).wait()
        gather(i + 2, slot).wait()

    def body(k, carry):
        step(2 * k, 0)
        step(2 * k + 1, 1)
        return carry

    lax.fori_loop(0, (NCHUNKS - 2) // 2, body, 0)

    last = (NCHUNKS - 1) % 2
    put(NCHUNKS - 1, last)
    pltpu.make_async_copy(rows_v[1 - last], out_hbm.at[pl.ds(0, CHUNK)],
                          ssem[1 - last]).wait()
    pltpu.make_async_copy(rows_v[last], out_hbm.at[pl.ds(0, CHUNK)],
                          ssem[last]).wait()


VOCAB = 100000
FUSE_BLOCK = 1000


def _fuse_body(w_ref, d_ref, out_ref):
    out_ref[:, :WORD_DIM] = w_ref[...]
    out_ref[:, WORD_DIM:] = d_ref[...]


_fuse_tables = pl.pallas_call(
    _fuse_body,
    grid=(VOCAB // FUSE_BLOCK,),
    in_specs=[
        pl.BlockSpec((FUSE_BLOCK, WORD_DIM), lambda i: (i, 0)),
        pl.BlockSpec((FUSE_BLOCK, DOMAIN_DIM), lambda i: (i, 0)),
    ],
    out_specs=pl.BlockSpec((FUSE_BLOCK, OUT_DIM), lambda i: (i, 0)),
    out_shape=jax.ShapeDtypeStruct((VOCAB, OUT_DIM), jnp.float32),
)


def kernel(word_table, domain_table, x):
    fused = _fuse_tables(word_table, domain_table)
    idx = x.reshape(-1).astype(jnp.int32)
    out = _fused_gather(fused, idx)
    return out.reshape(x.shape[0], x.shape[1], OUT_DIM)


# speedup vs baseline: 1.3634x; 1.0120x over previous
import functools

import jax
import jax.numpy as jnp
from jax import lax
from jax.experimental import pallas as pl
from jax.experimental.pallas import tpu as pltpu
from jax.experimental.pallas import tpu_sc as plsc

VOCAB = 100000
WORD_DIM = 300
DOMAIN_DIM = 100
OUT_DIM = WORD_DIM + DOMAIN_DIM
B = 1024
L = 200
N = B * L
NUM_WORKERS = 32
N_PER_W = N // NUM_WORKERS
CHUNK = 80
NCHUNKS = N_PER_W // CHUNK

FUSE_BLOCK = 512


def _fuse_body(wt_ref, dt_ref, q0_ref, q1_ref, q2_ref, q3_ref):
    q0_ref[...] = wt_ref[pl.ds(0, 128), :].T
    q1_ref[...] = wt_ref[pl.ds(128, 128), :].T
    q2_ref[:, :44] = wt_ref[pl.ds(256, 44), :].T
    q2_ref[:, 44:] = dt_ref[pl.ds(0, 84), :].T
    q3_ref[:, :16] = dt_ref[pl.ds(84, 16), :].T


_fuse_tables = pl.pallas_call(
    _fuse_body,
    grid=(pl.cdiv(VOCAB, FUSE_BLOCK),),
    in_specs=[
        pl.BlockSpec((WORD_DIM, FUSE_BLOCK), lambda i: (0, i)),
        pl.BlockSpec((DOMAIN_DIM, FUSE_BLOCK), lambda i: (0, i)),
    ],
    out_specs=[pl.BlockSpec((FUSE_BLOCK, 128), lambda i: (i, 0))] * 4,
    out_shape=[jax.ShapeDtypeStruct((VOCAB, 128), jnp.float32)] * 4,
)

_mesh = plsc.VectorSubcoreMesh(core_axis_name="c", subcore_axis_name="s")


@functools.partial(
    pl.kernel,
    mesh=_mesh,
    out_type=[jax.ShapeDtypeStruct((N, 128), jnp.float32)] * 4,
    compiler_params=pltpu.CompilerParams(use_tc_tiling_on_sc=False),
    scratch_types=[
        [pltpu.VMEM((CHUNK,), jnp.int32)] * 2,
        [[pltpu.VMEM((CHUNK, 128), jnp.float32)] * 4] * 2,
        [pltpu.SemaphoreType.DMA] * 2,
        [pltpu.SemaphoreType.DMA] * 2,
    ],
)
def _fused_gather(q0_hbm, q1_hbm, q2_hbm, q3_hbm, idx_hbm,
                  o0_hbm, o1_hbm, o2_hbm, o3_hbm,
                  idx_v, rows_v, gsem, ssem):
    wid = lax.axis_index("s") * 2 + lax.axis_index("c")
    base0 = wid * N_PER_W
    tabs = (q0_hbm, q1_hbm, q2_hbm, q3_hbm)
    outs = (o0_hbm, o1_hbm, o2_hbm, o3_hbm)

    def gather_wait(i, slot):
        base = base0 + i * CHUNK
        pltpu.sync_copy(idx_hbm.at[pl.ds(base, CHUNK)], idx_v[slot])
        cps = [pltpu.async_copy(tabs[q].at[idx_v[slot]], rows_v[slot][q],
                                gsem[slot]) for q in range(4)]
        for cp in cps:
            cp.wait()

    def put(i, slot):
        base = base0 + i * CHUNK
        for q in range(4):
            pltpu.async_copy(rows_v[slot][q], outs[q].at[pl.ds(base, CHUNK)],
                             ssem[slot])

    def drain_put(slot):
        for q in range(4):
            pltpu.make_async_copy(rows_v[slot][q],
                                  outs[q].at[pl.ds(0, CHUNK)],
                                  ssem[slot]).wait()

    gather_wait(0, 0)
    put(0, 0)
    gather_wait(1, 1)

    def step(i, slot):
        put(i + 1, 1 - slot)
        drain_put(slot)
        gather_wait(i + 2, slot)

    def body(k, carry):
        step(2 * k, 0)
        step(2 * k + 1, 1)
        return carry

    lax.fori_loop(0, (NCHUNKS - 2) // 2, body, 0)

    last = (NCHUNKS - 1) % 2
    put(NCHUNKS - 1, last)
    drain_put(1 - last)
    drain_put(last)


def _xpose_body(g0_ref, g1_ref, g2_ref, g3_ref, out_ref):
    out_ref[0, pl.ds(0, 128), :] = g0_ref[...].T
    out_ref[0, pl.ds(128, 128), :] = g1_ref[...].T
    out_ref[0, pl.ds(256, 128), :] = g2_ref[...].T
    out_ref[0, pl.ds(384, 16), :] = g3_ref[:, pl.ds(0, 16)].T


_xpose_out = pl.pallas_call(
    _xpose_body,
    grid=(L, B // 128),
    in_specs=[
        pl.BlockSpec((128, 128), lambda l, tb: (8 * l + tb, 0)),
    ] * 4,
    out_specs=pl.BlockSpec((1, OUT_DIM, 128), lambda l, tb: (l, 0, tb)),
    out_shape=jax.ShapeDtypeStruct((L, OUT_DIM, B), jnp.float32),
)


def kernel(word_table, domain_table, x):
    q0, q1, q2, q3 = _fuse_tables(word_table.T, domain_table.T)
    idx = x.T.reshape(-1).astype(jnp.int32)
    g0, g1, g2, g3 = _fused_gather(q0, q1, q2, q3, idx)
    out3 = _xpose_out(g0, g1, g2, g3)
    return out3.transpose(2, 0, 1)
